# Initial kernel scaffold; baseline (speedup 1.0000x reference)
import jax
import jax.numpy as jnp
from jax.experimental import pallas as pl


def kernel(idx, targets, table):
    raise NotImplementedError("write your pallas kernel here")


# speedup vs baseline: 1.4157x; 1.4157x over previous
import functools

import jax
import jax.numpy as jnp
from jax import lax
from jax.experimental import pallas as pl
from jax.experimental.pallas import tpu as pltpu
from jax.experimental.pallas import tpu_sc as plsc

V = 1000
N = 1024 * 50
VP = 1008

_info = plsc.get_sparse_core_info()
_NC, _NS = _info.num_cores, _info.num_subcores
NW = _NC * _NS
BPW = N // NW
RCH = 64
NRC = BPW // RCH
NG16 = BPW // 16

_mesh = plsc.VectorSubcoreMesh(core_axis_name="c", subcore_axis_name="s")


def _sc_body(idx_hbm, tgt_hbm, table_hbm,
             out_hbm, hist_hbm, s2p_hbm,
             idx_v, tgt_v, rows_v, hist_v, acc_v,
             sem_g):
  wid = lax.axis_index("s") * _NC + lax.axis_index("c")
  base = wid * BPW
  pltpu.sync_copy(idx_hbm.at[pl.ds(base, BPW)], idx_v)
  pltpu.sync_copy(tgt_hbm.at[pl.ds(base, BPW)], tgt_v)

  zz = jnp.zeros((16,), jnp.float32)

  def zbody(i, c):
    hist_v[pl.ds(i * 16, 16)] = zz
    return c
  lax.fori_loop(0, VP // 16, zbody, 0)
  acc_v[...] = zz

  ones = jnp.ones((16,), jnp.float32)

  def hbody(i, c):
    plsc.addupdate_scatter(hist_v, [idx_v[pl.ds(i * 16, 16)]], ones)
    return c
  lax.fori_loop(0, NG16, hbody, 0)

  pltpu.sync_copy(hist_v, hist_hbm.at[wid])

  def rbody(c, carry):
    pltpu.async_copy(
        table_hbm.at[idx_v.at[pl.ds(c * RCH, RCH)]], rows_v, sem_g).wait()
    pltpu.sync_copy(rows_v, out_hbm.at[pl.ds(base + c * RCH, RCH)])
    for g in range(RCH // 16):
      rr = jnp.arange(16, dtype=jnp.int32) + (g * 16)
      tt = tgt_v[pl.ds(c * RCH + g * 16, 16)]
      acc_v[...] = acc_v[...] + plsc.load_gather(rows_v, [rr, tt])
    return carry
  lax.fori_loop(0, NRC, rbody, 0)

  pltpu.sync_copy(acc_v, s2p_hbm.at[wid])


_sc_gather = functools.partial(
    pl.kernel,
    out_type=[
        jax.ShapeDtypeStruct((N, V), jnp.float32),
        jax.ShapeDtypeStruct((NW, VP), jnp.float32),
        jax.ShapeDtypeStruct((NW, 16), jnp.float32),
    ],
    mesh=_mesh,
    compiler_params=pltpu.CompilerParams(
        needs_layout_passes=False, use_tc_tiling_on_sc=False),
    scratch_types=[
        pltpu.VMEM((BPW,), jnp.int32),
        pltpu.VMEM((BPW,), jnp.int32),
        pltpu.VMEM((RCH, V), jnp.float32),
        pltpu.VMEM((VP,), jnp.float32),
        pltpu.VMEM((16,), jnp.float32),
        pltpu.SemaphoreType.DMA,
    ],
)(_sc_body)


def _tc_body(table_ref, hist_ref, s2p_ref, loss_ref):
  tab = table_ref[...]
  m = jnp.max(tab, axis=1, keepdims=True)
  se = jnp.sum(jnp.exp(tab - m), axis=1, keepdims=True)
  lse = m + jnp.log(se)
  cnt = jnp.sum(hist_ref[...], axis=0, keepdims=True)[:, :V]
  tot = lax.dot_general(cnt, lse, (((1,), (0,)), ((), ())),
                        preferred_element_type=jnp.float32,
                        precision=lax.Precision.HIGHEST)
  s2 = jnp.sum(s2p_ref[...])
  loss_ref[...] = (tot - s2) * (1.0 / N)


_tc_loss = pl.pallas_call(
    _tc_body,
    out_shape=jax.ShapeDtypeStruct((1, 1), jnp.float32),
)


def kernel(idx, targets, table):
  b, s = idx.shape
  logits_flat, hist, s2p = _sc_gather(
      idx.reshape(-1), targets.reshape(-1), table)
  loss = _tc_loss(table, hist, s2p)
  return logits_flat.reshape(b, s, V), loss[0, 0]


# speedup vs baseline: 1.4419x; 1.0185x over previous
import functools

import jax
import jax.numpy as jnp
from jax import lax
from jax.experimental import pallas as pl
from jax.experimental.pallas import tpu as pltpu
from jax.experimental.pallas import tpu_sc as plsc

V = 1000
N = 1024 * 50
VP = 1008

_info = plsc.get_sparse_core_info()
_NC, _NS = _info.num_cores, _info.num_subcores
NW = _NC * _NS
BPW = N // NW
RCH = 40
NRC = BPW // RCH
NG16 = BPW // 16

_mesh = plsc.VectorSubcoreMesh(core_axis_name="c", subcore_axis_name="s")


def _sc_body(idx_hbm, tgt_hbm, table_hbm,
             out_hbm, hist_hbm, s2p_hbm,
             idx_v, tgt_v, rows0_v, rows1_v, hist_v, acc_v,
             semg0, semg1, sems0, sems1):
  wid = lax.axis_index("s") * _NC + lax.axis_index("c")
  base = wid * BPW
  pltpu.sync_copy(idx_hbm.at[pl.ds(base, BPW)], idx_v)
  pltpu.sync_copy(tgt_hbm.at[pl.ds(base, BPW)], tgt_v)

  zz = jnp.zeros((16,), jnp.float32)

  def zbody(i, c):
    hist_v[pl.ds(i * 16, 16)] = zz
    return c
  lax.fori_loop(0, VP // 16, zbody, 0)
  acc_v[...] = zz

  ones = jnp.ones((16,), jnp.float32)

  def hbody(i, c):
    plsc.addupdate_scatter(hist_v, [idx_v[pl.ds(i * 16, 16)]], ones)
    return c
  lax.fori_loop(0, NG16, hbody, 0)

  pltpu.sync_copy(hist_v, hist_hbm.at[wid])

  def issue_gather(c, buf, semg):
    pltpu.async_copy(table_hbm.at[idx_v.at[pl.ds(c * RCH, RCH)]], buf, semg)

  def wait_gather(c, buf, semg):
    pltpu.make_async_copy(
        table_hbm.at[idx_v.at[pl.ds(c * RCH, RCH)]], buf, semg).wait()

  def issue_scatter(c, buf, sems):
    pltpu.async_copy(buf, out_hbm.at[pl.ds(base + c * RCH, RCH)], sems)

  def wait_scatter(c, buf, sems):
    pltpu.make_async_copy(
        buf, out_hbm.at[pl.ds(base + c * RCH, RCH)], sems).wait()

  def consume(c, buf):
    for g in range(RCH // 16):
      rr = jnp.arange(16, dtype=jnp.int32) + (g * 16)
      tt = tgt_v[pl.ds(c * RCH + g * 16, 16)]
      acc_v[...] = acc_v[...] + plsc.load_gather(buf, [rr, tt])

  def step(k, buf, semg, sems, obuf, osemg, osems, first, last):
    if not first:
      wait_scatter(k - 1, obuf, osems)
    if not last:
      issue_gather(k + 1, obuf, osemg)
    wait_gather(k, buf, semg)
    issue_scatter(k, buf, sems)
    consume(k, buf)

  issue_gather(0, rows0_v, semg0)
  step(0, rows0_v, semg0, sems0, rows1_v, semg1, sems1, True, False)

  def pbody(p, carry):
    k = 2 * p - 1
    step(k, rows1_v, semg1, sems1, rows0_v, semg0, sems0, False, False)
    step(k + 1, rows0_v, semg0, sems0, rows1_v, semg1, sems1, False, False)
    return carry
  lax.fori_loop(1, NRC // 2, pbody, 0)

  step(NRC - 1, rows1_v, semg1, sems1, rows0_v, semg0, sems0, False, True)
  wait_scatter(NRC - 1, rows1_v, sems1)

  pltpu.sync_copy(acc_v, s2p_hbm.at[wid])


_sc_gather = functools.partial(
    pl.kernel,
    out_type=[
        jax.ShapeDtypeStruct((N, V), jnp.float32),
        jax.ShapeDtypeStruct((NW, VP), jnp.float32),
        jax.ShapeDtypeStruct((NW, 16), jnp.float32),
    ],
    mesh=_mesh,
    compiler_params=pltpu.CompilerParams(
        needs_layout_passes=False, use_tc_tiling_on_sc=False),
    scratch_types=[
        pltpu.VMEM((BPW,), jnp.int32),
        pltpu.VMEM((BPW,), jnp.int32),
        pltpu.VMEM((RCH, V), jnp.float32),
        pltpu.VMEM((RCH, V), jnp.float32),
        pltpu.VMEM((VP,), jnp.float32),
        pltpu.VMEM((16,), jnp.float32),
        pltpu.SemaphoreType.DMA,
        pltpu.SemaphoreType.DMA,
        pltpu.SemaphoreType.DMA,
        pltpu.SemaphoreType.DMA,
    ],
)(_sc_body)


def _tc_body(table_ref, hist_ref, s2p_ref, loss_ref):
  tab = table_ref[...]
  m = jnp.max(tab, axis=1, keepdims=True)
  se = jnp.sum(jnp.exp(tab - m), axis=1, keepdims=True)
  lse = m + jnp.log(se)
  cnt = jnp.sum(hist_ref[...], axis=0, keepdims=True)[:, :V]
  tot = lax.dot_general(cnt, lse, (((1,), (0,)), ((), ())),
                        preferred_element_type=jnp.float32,
                        precision=lax.Precision.HIGHEST)
  s2 = jnp.sum(s2p_ref[...])
  loss_ref[...] = (tot - s2) * (1.0 / N)


_tc_loss = pl.pallas_call(
    _tc_body,
    out_shape=jax.ShapeDtypeStruct((1, 1), jnp.float32),
)


def kernel(idx, targets, table):
  b, s = idx.shape
  logits_flat, hist, s2p = _sc_gather(
      idx.reshape(-1), targets.reshape(-1), table)
  loss = _tc_loss(table, hist, s2p)
  return logits_flat.reshape(b, s, V), loss[0, 0]
